# Initial kernel scaffold; baseline (speedup 1.0000x reference)
import functools

import jax
import jax.numpy as jnp
from jax import lax
from jax.experimental import pallas as pl
from jax.experimental.pallas import tpu as pltpu
from jax.experimental.pallas import tpu_sc as plsc

N = 10000
E = 320000
F = 128

NC = 2
NS = 16
NW = NC * NS
EPW = E // NW
CH = 125
NCHUNK = EPW // CH
RPT = N // NS
RCH = RPT // CH

_SELU_ALPHA = 1.6732632423543772
_SELU_SCALE = 1.0507009873554805

_mesh = plsc.VectorSubcoreMesh(core_axis_name="c", subcore_axis_name="s")


def _zero_rows(buf, rows):
    zeros = jnp.zeros((16,), jnp.float32)

    def body(i, _):
        r = i // (F // 16)
        l = i % (F // 16)
        buf[r, pl.ds(l * 16, 16)] = zeros
        return 0

    lax.fori_loop(0, rows * (F // 16), body, 0)


@functools.partial(
    pl.kernel,
    out_type=jax.ShapeDtypeStruct((NW, N), jnp.float32),
    mesh=_mesh,
    scratch_types=[
        pltpu.VMEM((EPW // 16, 16), jnp.int32),
        pltpu.VMEM((N,), jnp.float32),
    ],
)
def _hist_kernel(dst_hbm, out_hbm, dst_v, hist_v):
    c = lax.axis_index("c")
    s = lax.axis_index("s")
    wid = s * NC + c
    pltpu.sync_copy(dst_hbm.at[wid], dst_v)

    zeros = jnp.zeros((16,), jnp.float32)

    def zbody(i, _):
        hist_v[pl.ds(i * 16, 16)] = zeros
        return 0

    lax.fori_loop(0, N // 16, zbody, 0)

    ones = jnp.ones((16,), jnp.float32)

    def body(i, _):
        idx = dst_v[i, :]
        plsc.addupdate_scatter(hist_v, [idx], ones)
        return 0

    lax.fori_loop(0, EPW // 16, body, 0)
    pltpu.sync_copy(hist_v, out_hbm.at[wid])


@functools.partial(
    pl.kernel,
    out_type=jax.ShapeDtypeStruct((NC, N, F), jnp.float32),
    mesh=_mesh,
    scratch_types=[
        pltpu.VMEM((NCHUNK, CH), jnp.int32),
        pltpu.VMEM((NCHUNK, CH), jnp.int32),
        pltpu.VMEM((CH, F), jnp.float32),
        pltpu.VMEM_SHARED((N, F), jnp.float32),
    ],
)
def _prop_kernel(xt_hbm, src_hbm, dst_hbm, out_hbm, src_v, dst_v, rows_v, acc):
    c = lax.axis_index("c")
    s = lax.axis_index("s")
    wid = s * NC + c

    pltpu.sync_copy(src_hbm.at[wid], src_v)
    pltpu.sync_copy(dst_hbm.at[wid], dst_v)

    _zero_rows(rows_v, CH)
    for t in range(RCH):
        pltpu.sync_copy(rows_v, acc.at[pl.ds(s * RPT + t * CH, CH)])
    plsc.subcore_barrier()

    def chunk(j, _):
        pltpu.sync_copy(xt_hbm.at[src_v.at[j]], rows_v)
        pltpu.sync_copy(rows_v, acc.at[dst_v.at[j]], add=True)
        return 0

    lax.fori_loop(0, NCHUNK, chunk, 0)
    plsc.subcore_barrier()

    for t in range(RCH):
        r0 = s * RPT + t * CH
        pltpu.sync_copy(acc.at[pl.ds(r0, CH)], out_hbm.at[c, pl.ds(r0, CH)])


def _prep_body(hist_ref, x_ref, dinv_ref, xt_ref):
    deg = jnp.sum(jnp.transpose(hist_ref[...]), axis=1, keepdims=True)
    dinv = jnp.where(deg > 0.0, lax.rsqrt(jnp.maximum(deg, 1e-12)), 0.0)
    dinv_ref[...] = dinv
    xt_ref[...] = x_ref[...] * dinv


def _merge_body(x_ref, p_ref, dinv_ref, h_ref, xt_ref):
    dinv = dinv_ref[...]
    h = x_ref[...] + dinv * (p_ref[0] + p_ref[1])
    h_ref[...] = h
    xt_ref[...] = dinv * h


def _mid_body(h1_ref, p_ref, dinv_ref, w1_ref, b1_ref, g_ref, beta_ref,
              h3_ref, xt_ref):
    dinv = dinv_ref[...]
    h2 = h1_ref[...] + dinv * (p_ref[0] + p_ref[1])
    z = jnp.dot(h2, w1_ref[...], preferred_element_type=jnp.float32,
                precision=lax.Precision.HIGHEST) + b1_ref[...]
    mean = jnp.mean(z, axis=0, keepdims=True)
    var = jnp.mean(jnp.square(z - mean), axis=0, keepdims=True)
    zn = (z - mean) * lax.rsqrt(var + 1e-5) * g_ref[...] + beta_ref[...]
    h3 = _SELU_SCALE * jnp.where(
        zn > 0.0, zn, _SELU_ALPHA * (jnp.exp(jnp.minimum(zn, 0.0)) - 1.0))
    h3_ref[...] = h3
    xt_ref[...] = dinv * h3


def _out_body(h3_ref, p_ref, dinv_ref, w2_ref, b2_ref, o_ref):
    h4 = h3_ref[...] + dinv_ref[...] * (p_ref[0] + p_ref[1])
    z = jnp.dot(h4, w2_ref[...], preferred_element_type=jnp.float32,
                precision=lax.Precision.HIGHEST) + b2_ref[...]
    z = z - jnp.max(z, axis=1, keepdims=True)
    ez = jnp.exp(z)
    o_ref[...] = ez / jnp.sum(ez, axis=1, keepdims=True)


def kernel(x, edge_index, W1, b1, gamma, beta, W2, b2):
    src = edge_index[0].astype(jnp.int32)
    dst = edge_index[1].astype(jnp.int32)
    src3 = src.reshape(NW, NCHUNK, CH)
    dst3 = dst.reshape(NW, NCHUNK, CH)
    dst16 = dst.reshape(NW, EPW // 16, 16)

    f32 = jnp.float32
    hist = _hist_kernel(dst16)

    dinv, xt1 = pl.pallas_call(
        _prep_body,
        out_shape=(jax.ShapeDtypeStruct((N, 1), f32),
                   jax.ShapeDtypeStruct((N, F), f32)),
    )(hist, x)

    p1 = _prop_kernel(xt1, src3, dst3)
    h1, xt2 = pl.pallas_call(
        _merge_body,
        out_shape=(jax.ShapeDtypeStruct((N, F), f32),
                   jax.ShapeDtypeStruct((N, F), f32)),
    )(x, p1, dinv)

    p2 = _prop_kernel(xt2, src3, dst3)
    h3, xt3 = pl.pallas_call(
        _mid_body,
        out_shape=(jax.ShapeDtypeStruct((N, F), f32),
                   jax.ShapeDtypeStruct((N, F), f32)),
    )(h1, p2, dinv, W1, b1.reshape(1, F), gamma.reshape(1, F),
      beta.reshape(1, F))

    p3 = _prop_kernel(xt3, src3, dst3)
    out = pl.pallas_call(
        _out_body,
        out_shape=jax.ShapeDtypeStruct((N, F), f32),
    )(h3, p3, dinv, W2, b2.reshape(1, F))
    return out


# speedup vs baseline: 19.1207x; 19.1207x over previous
import functools

import jax
import jax.numpy as jnp
from jax import lax
from jax.experimental import pallas as pl
from jax.experimental.pallas import tpu as pltpu
from jax.experimental.pallas import tpu_sc as plsc

N = 10000
E = 320000
F = 128

NC = 2
NS = 16
NW = NC * NS
EPW = E // NW
CH = 125
NCHUNK = EPW // CH
RPT = 624

_SELU_ALPHA = 1.6732632423543772
_SELU_SCALE = 1.0507009873554805

_mesh = plsc.VectorSubcoreMesh(core_axis_name="c", subcore_axis_name="s")
_sc_params = pltpu.CompilerParams(needs_layout_passes=False)


def _zero_rows(buf, rows):
    zeros = jnp.zeros((16,), jnp.float32)

    def body(i, _):
        r = i // (F // 16)
        l = i % (F // 16)
        buf[r, pl.ds(l * 16, 16)] = zeros
        return 0

    lax.fori_loop(0, rows * (F // 16), body, 0)


@functools.partial(
    pl.kernel,
    out_type=jax.ShapeDtypeStruct((NW, 1, N), jnp.float32),
    mesh=_mesh,
    compiler_params=_sc_params,
    scratch_types=[
        pltpu.VMEM((EPW // 16, 16), jnp.int32),
        pltpu.VMEM((N,), jnp.float32),
    ],
)
def _hist_kernel(dst_hbm, out_hbm, dst_v, hist_v):
    c = lax.axis_index("c")
    s = lax.axis_index("s")
    wid = s * NC + c
    pltpu.sync_copy(dst_hbm.at[wid], dst_v)

    zeros = jnp.zeros((16,), jnp.float32)

    def zbody(i, _):
        hist_v[pl.ds(i * 16, 16)] = zeros
        return 0

    lax.fori_loop(0, N // 16, zbody, 0)

    ones = jnp.ones((16,), jnp.float32)

    def body(i, _):
        idx = dst_v[i, :]
        plsc.addupdate_scatter(hist_v, [idx], ones)
        return 0

    lax.fori_loop(0, EPW // 16, body, 0)
    pltpu.sync_copy(hist_v, out_hbm.at[wid, 0])


@functools.partial(
    pl.kernel,
    out_type=jax.ShapeDtypeStruct((NC, N, F), jnp.float32),
    mesh=_mesh,
    compiler_params=_sc_params,
    scratch_types=[
        pltpu.VMEM((NCHUNK, CH), jnp.int32),
        pltpu.VMEM((NCHUNK, CH), jnp.int32),
        pltpu.VMEM((CH, F), jnp.float32),
        pltpu.VMEM_SHARED((N, F), jnp.float32),
    ],
)
def _prop_kernel(xt_hbm, src_hbm, dst_hbm, out_hbm,
                 src_v, dst_v, rows_v, acc):
    c = lax.axis_index("c")
    s = lax.axis_index("s")
    wid = s * NC + c

    pltpu.sync_copy(src_hbm.at[wid], src_v)
    pltpu.sync_copy(dst_hbm.at[wid], dst_v)

    _zero_rows(rows_v, CH)
    base = s * RPT
    for t in range(5):
        pltpu.sync_copy(rows_v.at[pl.ds(0, 120)],
                        acc.at[pl.ds(base + t * 120, 120)])
    pltpu.sync_copy(rows_v.at[pl.ds(0, 24)], acc.at[pl.ds(base + 600, 24)])

    @pl.when(s == NS - 1)
    def _():
        pltpu.sync_copy(rows_v.at[pl.ds(0, N - NS * RPT)],
                        acc.at[pl.ds(NS * RPT, N - NS * RPT)])

    plsc.subcore_barrier()

    def chunk(j, _):
        pltpu.sync_copy(xt_hbm.at[src_v.at[j]], rows_v)
        pltpu.sync_copy(rows_v, acc.at[dst_v.at[j]], add=True)
        return 0

    lax.fori_loop(0, NCHUNK, chunk, 0)
    plsc.subcore_barrier()

    pltpu.sync_copy(acc.at[pl.ds(base, RPT)], out_hbm.at[c, pl.ds(base, RPT)])

    @pl.when(s == NS - 1)
    def _():
        pltpu.sync_copy(acc.at[pl.ds(NS * RPT, N - NS * RPT)],
                        out_hbm.at[c, pl.ds(NS * RPT, N - NS * RPT)])


def _prep_body(hist_ref, x_ref, dinv_ref, xt_ref):
    deg = jnp.sum(jnp.transpose(hist_ref[...]), axis=1, keepdims=True)
    dinv = jnp.where(deg > 0.0, lax.rsqrt(jnp.maximum(deg, 1e-12)), 0.0)
    dinv_ref[...] = dinv
    xt_ref[...] = x_ref[...] * dinv


def _merge_body(x_ref, p_ref, dinv_ref, h_ref, xt_ref):
    dinv = dinv_ref[...]
    h = x_ref[...] + dinv * (p_ref[0] + p_ref[1])
    h_ref[...] = h
    xt_ref[...] = dinv * h


def _mid_body(h1_ref, p_ref, dinv_ref, w1_ref, b1_ref, g_ref, beta_ref,
              h3_ref, xt_ref):
    dinv = dinv_ref[...]
    h2 = h1_ref[...] + dinv * (p_ref[0] + p_ref[1])
    z = jnp.dot(h2, w1_ref[...], preferred_element_type=jnp.float32,
                precision=lax.Precision.HIGHEST) + b1_ref[...]
    mean = jnp.mean(z, axis=0, keepdims=True)
    var = jnp.mean(jnp.square(z - mean), axis=0, keepdims=True)
    zn = (z - mean) * lax.rsqrt(var + 1e-5) * g_ref[...] + beta_ref[...]
    h3 = _SELU_SCALE * jnp.where(
        zn > 0.0, zn, _SELU_ALPHA * (jnp.exp(jnp.minimum(zn, 0.0)) - 1.0))
    h3_ref[...] = h3
    xt_ref[...] = dinv * h3


def _out_body(h3_ref, p_ref, dinv_ref, w2_ref, b2_ref, o_ref):
    h4 = h3_ref[...] + dinv_ref[...] * (p_ref[0] + p_ref[1])
    z = jnp.dot(h4, w2_ref[...], preferred_element_type=jnp.float32,
                precision=lax.Precision.HIGHEST) + b2_ref[...]
    z = z - jnp.max(z, axis=1, keepdims=True)
    ez = jnp.exp(z)
    o_ref[...] = ez / jnp.sum(ez, axis=1, keepdims=True)


def kernel(x, edge_index, W1, b1, gamma, beta, W2, b2):
    src = edge_index[0].astype(jnp.int32)
    dst = edge_index[1].astype(jnp.int32)
    src3 = src.reshape(NW, NCHUNK, CH)
    dst3 = dst.reshape(NW, NCHUNK, CH)
    dst16 = dst.reshape(NW, EPW // 16, 16)

    f32 = jnp.float32
    hist = _hist_kernel(dst16).reshape(NW, N)

    dinv, xt1 = pl.pallas_call(
        _prep_body,
        out_shape=(jax.ShapeDtypeStruct((N, 1), f32),
                   jax.ShapeDtypeStruct((N, F), f32)),
    )(hist, x)

    p1 = _prop_kernel(xt1, src3, dst3)
    h1, xt2 = pl.pallas_call(
        _merge_body,
        out_shape=(jax.ShapeDtypeStruct((N, F), f32),
                   jax.ShapeDtypeStruct((N, F), f32)),
    )(x, p1, dinv)

    p2 = _prop_kernel(xt2, src3, dst3)
    h3, xt3 = pl.pallas_call(
        _mid_body,
        out_shape=(jax.ShapeDtypeStruct((N, F), f32),
                   jax.ShapeDtypeStruct((N, F), f32)),
    )(h1, p2, dinv, W1, b1.reshape(1, F), gamma.reshape(1, F),
      beta.reshape(1, F))

    p3 = _prop_kernel(xt3, src3, dst3)
    out = pl.pallas_call(
        _out_body,
        out_shape=jax.ShapeDtypeStruct((N, F), f32),
    )(h3, p3, dinv, W2, b2.reshape(1, F))
    return out


# speedup vs baseline: 27.7158x; 1.4495x over previous
import functools

import jax
import jax.numpy as jnp
from jax import lax
from jax.experimental import pallas as pl
from jax.experimental.pallas import tpu as pltpu
from jax.experimental.pallas import tpu_sc as plsc

N = 10000
E = 320000
F = 128

NC = 2
NS = 16
NW = NC * NS
EPW = E // NW
CH = 125
NCHUNK = EPW // CH
HALF = NCHUNK // 2
RPT = 624

_SELU_ALPHA = 1.6732632423543772
_SELU_SCALE = 1.0507009873554805

_mesh = plsc.VectorSubcoreMesh(core_axis_name="c", subcore_axis_name="s")
_sc_params = pltpu.CompilerParams(needs_layout_passes=False)


def _zero_rows(buf, rows):
    zeros = jnp.zeros((16,), jnp.float32)

    def body(i, _):
        r = i // (F // 16)
        l = i % (F // 16)
        buf[r, pl.ds(l * 16, 16)] = zeros
        return 0

    lax.fori_loop(0, rows * (F // 16), body, 0)


@functools.partial(
    pl.kernel,
    out_type=jax.ShapeDtypeStruct((NW, 1, N), jnp.float32),
    mesh=_mesh,
    compiler_params=_sc_params,
    scratch_types=[
        pltpu.VMEM((EPW // 16, 16), jnp.int32),
        pltpu.VMEM((N,), jnp.float32),
    ],
)
def _hist_kernel(dst_hbm, out_hbm, dst_v, hist_v):
    c = lax.axis_index("c")
    s = lax.axis_index("s")
    wid = s * NC + c
    pltpu.sync_copy(dst_hbm.at[wid], dst_v)

    zeros = jnp.zeros((16,), jnp.float32)

    def zbody(i, _):
        hist_v[pl.ds(i * 16, 16)] = zeros
        return 0

    lax.fori_loop(0, N // 16, zbody, 0)

    ones = jnp.ones((16,), jnp.float32)

    def body(i, _):
        idx = dst_v[i, :]
        plsc.addupdate_scatter(hist_v, [idx], ones)
        return 0

    lax.fori_loop(0, EPW // 16, body, 0)
    pltpu.sync_copy(hist_v, out_hbm.at[wid, 0])


@functools.partial(
    pl.kernel,
    out_type=jax.ShapeDtypeStruct((NC, N, F), jnp.float32),
    mesh=_mesh,
    compiler_params=_sc_params,
    scratch_types=[
        pltpu.VMEM((HALF, CH), jnp.int32),
        pltpu.VMEM((HALF, CH), jnp.int32),
        pltpu.VMEM((CH, F), jnp.float32),
        pltpu.VMEM((CH, F), jnp.float32),
        pltpu.SemaphoreType.DMA,
        pltpu.SemaphoreType.DMA,
        pltpu.VMEM_SHARED((N, F), jnp.float32),
    ],
)
def _prop_kernel(xt_hbm, src_hbm, dst_hbm, out_hbm,
                 src_v, dst_v, rows_a, rows_b, sem_a, sem_b, acc):
    c = lax.axis_index("c")
    s = lax.axis_index("s")
    wid = s * NC + c

    _zero_rows(rows_a, CH)
    base = s * RPT
    for t in range(5):
        pltpu.sync_copy(rows_a.at[pl.ds(0, 120)],
                        acc.at[pl.ds(base + t * 120, 120)])
    pltpu.sync_copy(rows_a.at[pl.ds(0, 24)], acc.at[pl.ds(base + 600, 24)])

    @pl.when(s == NS - 1)
    def _():
        pltpu.sync_copy(rows_a.at[pl.ds(0, N - NS * RPT)],
                        acc.at[pl.ds(NS * RPT, N - NS * RPT)])

    plsc.subcore_barrier()

    bufs = (rows_a, rows_b)
    sems = (sem_a, sem_b)

    for r in range(2):
        pltpu.sync_copy(src_hbm.at[wid, pl.ds(r * HALF, HALF)], src_v)
        pltpu.sync_copy(dst_hbm.at[wid, pl.ds(r * HALF, HALF)], dst_v)
        pltpu.async_copy(xt_hbm.at[src_v.at[0]], rows_a, sem_a)
        pltpu.async_copy(xt_hbm.at[src_v.at[1]], rows_b, sem_b)

        def pair(k, _):
            for b in range(2):
                j = 2 * k + b
                pltpu.make_async_copy(xt_hbm.at[src_v.at[j]],
                                      bufs[b], sems[b]).wait()
                pltpu.sync_copy(bufs[b], acc.at[dst_v.at[j]], add=True)
                pltpu.async_copy(xt_hbm.at[src_v.at[j + 2]], bufs[b], sems[b])
            return 0

        lax.fori_loop(0, HALF // 2 - 1, pair, 0)
        for b in range(2):
            j = HALF - 2 + b
            pltpu.make_async_copy(xt_hbm.at[src_v.at[j]],
                                  bufs[b], sems[b]).wait()
            pltpu.sync_copy(bufs[b], acc.at[dst_v.at[j]], add=True)

    plsc.subcore_barrier()

    pltpu.sync_copy(acc.at[pl.ds(base, RPT)], out_hbm.at[c, pl.ds(base, RPT)])

    @pl.when(s == NS - 1)
    def _():
        pltpu.sync_copy(acc.at[pl.ds(NS * RPT, N - NS * RPT)],
                        out_hbm.at[c, pl.ds(NS * RPT, N - NS * RPT)])


def _prep_body(hist_ref, x_ref, dinv_ref, xt_ref):
    deg = jnp.sum(jnp.transpose(hist_ref[...]), axis=1, keepdims=True)
    dinv = jnp.where(deg > 0.0, lax.rsqrt(jnp.maximum(deg, 1e-12)), 0.0)
    dinv_ref[...] = dinv
    xt_ref[...] = x_ref[...] * dinv


def _merge_body(x_ref, p_ref, dinv_ref, h_ref, xt_ref):
    dinv = dinv_ref[...]
    h = x_ref[...] + dinv * (p_ref[0] + p_ref[1])
    h_ref[...] = h
    xt_ref[...] = dinv * h


def _mid_body(h1_ref, p_ref, dinv_ref, w1_ref, b1_ref, g_ref, beta_ref,
              h3_ref, xt_ref):
    dinv = dinv_ref[...]
    h2 = h1_ref[...] + dinv * (p_ref[0] + p_ref[1])
    z = jnp.dot(h2, w1_ref[...], preferred_element_type=jnp.float32,
                precision=lax.Precision.HIGHEST) + b1_ref[...]
    mean = jnp.mean(z, axis=0, keepdims=True)
    var = jnp.mean(jnp.square(z - mean), axis=0, keepdims=True)
    zn = (z - mean) * lax.rsqrt(var + 1e-5) * g_ref[...] + beta_ref[...]
    h3 = _SELU_SCALE * jnp.where(
        zn > 0.0, zn, _SELU_ALPHA * (jnp.exp(jnp.minimum(zn, 0.0)) - 1.0))
    h3_ref[...] = h3
    xt_ref[...] = dinv * h3


def _out_body(h3_ref, p_ref, dinv_ref, w2_ref, b2_ref, o_ref):
    h4 = h3_ref[...] + dinv_ref[...] * (p_ref[0] + p_ref[1])
    z = jnp.dot(h4, w2_ref[...], preferred_element_type=jnp.float32,
                precision=lax.Precision.HIGHEST) + b2_ref[...]
    z = z - jnp.max(z, axis=1, keepdims=True)
    ez = jnp.exp(z)
    o_ref[...] = ez / jnp.sum(ez, axis=1, keepdims=True)


def kernel(x, edge_index, W1, b1, gamma, beta, W2, b2):
    src = edge_index[0].astype(jnp.int32)
    dst = edge_index[1].astype(jnp.int32)
    src3 = src.reshape(NW, NCHUNK, CH)
    dst3 = dst.reshape(NW, NCHUNK, CH)
    dst16 = dst.reshape(NW, EPW // 16, 16)

    f32 = jnp.float32
    hist = _hist_kernel(dst16).reshape(NW, N)

    dinv, xt1 = pl.pallas_call(
        _prep_body,
        out_shape=(jax.ShapeDtypeStruct((N, 1), f32),
                   jax.ShapeDtypeStruct((N, F), f32)),
    )(hist, x)

    p1 = _prop_kernel(xt1, src3, dst3)
    h1, xt2 = pl.pallas_call(
        _merge_body,
        out_shape=(jax.ShapeDtypeStruct((N, F), f32),
                   jax.ShapeDtypeStruct((N, F), f32)),
    )(x, p1, dinv)

    p2 = _prop_kernel(xt2, src3, dst3)
    h3, xt3 = pl.pallas_call(
        _mid_body,
        out_shape=(jax.ShapeDtypeStruct((N, F), f32),
                   jax.ShapeDtypeStruct((N, F), f32)),
    )(h1, p2, dinv, W1, b1.reshape(1, F), gamma.reshape(1, F),
      beta.reshape(1, F))

    p3 = _prop_kernel(xt3, src3, dst3)
    out = pl.pallas_call(
        _out_body,
        out_shape=jax.ShapeDtypeStruct((N, F), f32),
    )(h3, p3, dinv, W2, b2.reshape(1, F))
    return out


# speedup vs baseline: 29.1100x; 1.0503x over previous
import functools

import jax
import jax.numpy as jnp
from jax import lax
from jax.experimental import pallas as pl
from jax.experimental.pallas import tpu as pltpu
from jax.experimental.pallas import tpu_sc as plsc

N = 10000
E = 320000
F = 128

NC = 2
NS = 16
NW = NC * NS
EPW = E // NW
CH = 125
NCHUNK = EPW // CH
HALF = NCHUNK // 2
RPT = 624

_SELU_ALPHA = 1.6732632423543772
_SELU_SCALE = 1.0507009873554805

_mesh = plsc.VectorSubcoreMesh(core_axis_name="c", subcore_axis_name="s")
_sc_params = pltpu.CompilerParams(needs_layout_passes=False)


def _zero_rows(buf, rows):
    zeros = jnp.zeros((16,), jnp.float32)

    def body(i, _):
        r = i // (F // 16)
        l = i % (F // 16)
        buf[r, pl.ds(l * 16, 16)] = zeros
        return 0

    lax.fori_loop(0, rows * (F // 16), body, 0)


@functools.partial(
    pl.kernel,
    out_type=jax.ShapeDtypeStruct((NW, 1, N), jnp.float32),
    mesh=_mesh,
    compiler_params=_sc_params,
    scratch_types=[
        pltpu.VMEM((NCHUNK, CH), jnp.int32),
        pltpu.VMEM((N,), jnp.float32),
    ],
)
def _hist_kernel(edges_hbm, out_hbm, dst_v, hist_v):
    c = lax.axis_index("c")
    s = lax.axis_index("s")
    wid = s * NC + c
    pltpu.sync_copy(edges_hbm.at[1, wid], dst_v)

    zeros = jnp.zeros((16,), jnp.float32)

    def zbody(i, _):
        hist_v[pl.ds(i * 16, 16)] = zeros
        return 0

    lax.fori_loop(0, N // 16, zbody, 0)

    ones = jnp.ones((16,), jnp.float32)
    tail_mask = lax.iota(jnp.int32, 16) >= ((CH // 16) + 1) * 16 - CH

    def body(i, _):
        for l in range(CH // 16):
            idx = dst_v[i, pl.ds(l * 16, 16)]
            plsc.addupdate_scatter(hist_v, [idx], ones)
        idx = dst_v[i, pl.ds(CH - 16, 16)]
        plsc.addupdate_scatter(hist_v, [idx], ones, mask=tail_mask)
        return 0

    lax.fori_loop(0, NCHUNK, body, 0)
    pltpu.sync_copy(hist_v, out_hbm.at[wid, 0])


@functools.partial(
    pl.kernel,
    out_type=jax.ShapeDtypeStruct((NC, N, F), jnp.float32),
    mesh=_mesh,
    compiler_params=_sc_params,
    scratch_types=[
        pltpu.VMEM((HALF, CH), jnp.int32),
        pltpu.VMEM((HALF, CH), jnp.int32),
        pltpu.VMEM((CH, F), jnp.float32),
        pltpu.VMEM((CH, F), jnp.float32),
        pltpu.SemaphoreType.DMA,
        pltpu.SemaphoreType.DMA,
        pltpu.VMEM_SHARED((N, F), jnp.float32),
    ],
)
def _prop_kernel(xt_hbm, edges_hbm, out_hbm,
                 src_v, dst_v, rows_a, rows_b, sem_a, sem_b, acc):
    c = lax.axis_index("c")
    s = lax.axis_index("s")
    wid = s * NC + c

    _zero_rows(rows_a, CH)
    base = s * RPT
    for t in range(5):
        pltpu.sync_copy(rows_a.at[pl.ds(0, 120)],
                        acc.at[pl.ds(base + t * 120, 120)])
    pltpu.sync_copy(rows_a.at[pl.ds(0, 24)], acc.at[pl.ds(base + 600, 24)])

    @pl.when(s == NS - 1)
    def _():
        pltpu.sync_copy(rows_a.at[pl.ds(0, N - NS * RPT)],
                        acc.at[pl.ds(NS * RPT, N - NS * RPT)])

    plsc.subcore_barrier()

    bufs = (rows_a, rows_b)
    sems = (sem_a, sem_b)

    for r in range(2):
        pltpu.sync_copy(edges_hbm.at[0, wid, pl.ds(r * HALF, HALF)], src_v)
        pltpu.sync_copy(edges_hbm.at[1, wid, pl.ds(r * HALF, HALF)], dst_v)
        pltpu.async_copy(xt_hbm.at[src_v.at[0]], rows_a, sem_a)
        pltpu.async_copy(xt_hbm.at[src_v.at[1]], rows_b, sem_b)

        def pair(k, _):
            for b in range(2):
                j = 2 * k + b
                pltpu.make_async_copy(xt_hbm.at[src_v.at[j]],
                                      bufs[b], sems[b]).wait()
                pltpu.sync_copy(bufs[b], acc.at[dst_v.at[j]], add=True)
                pltpu.async_copy(xt_hbm.at[src_v.at[j + 2]], bufs[b], sems[b])
            return 0

        lax.fori_loop(0, HALF // 2 - 1, pair, 0)
        for b in range(2):
            j = HALF - 2 + b
            pltpu.make_async_copy(xt_hbm.at[src_v.at[j]],
                                  bufs[b], sems[b]).wait()
            pltpu.sync_copy(bufs[b], acc.at[dst_v.at[j]], add=True)

    plsc.subcore_barrier()

    pltpu.sync_copy(acc.at[pl.ds(base, RPT)], out_hbm.at[c, pl.ds(base, RPT)])

    @pl.when(s == NS - 1)
    def _():
        pltpu.sync_copy(acc.at[pl.ds(NS * RPT, N - NS * RPT)],
                        out_hbm.at[c, pl.ds(NS * RPT, N - NS * RPT)])


def _prep_body(hist_ref, x_ref, dinv_ref, xt_ref):
    deg = jnp.sum(jnp.transpose(hist_ref[:, 0, :]), axis=1, keepdims=True)
    dinv = jnp.where(deg > 0.0, lax.rsqrt(jnp.maximum(deg, 1e-12)), 0.0)
    dinv_ref[...] = dinv
    xt_ref[...] = x_ref[...] * dinv


def _merge_body(x_ref, p_ref, dinv_ref, h_ref, xt_ref):
    dinv = dinv_ref[...]
    h = x_ref[...] + dinv * (p_ref[0] + p_ref[1])
    h_ref[...] = h
    xt_ref[...] = dinv * h


def _mid_body(h1_ref, p_ref, dinv_ref, w1_ref, b1_ref, g_ref, beta_ref,
              h3_ref, xt_ref):
    dinv = dinv_ref[...]
    h2 = h1_ref[...] + dinv * (p_ref[0] + p_ref[1])
    z = jnp.dot(h2, w1_ref[...], preferred_element_type=jnp.float32,
                precision=lax.Precision.HIGHEST) + b1_ref[...]
    mean = jnp.mean(z, axis=0, keepdims=True)
    var = jnp.mean(jnp.square(z - mean), axis=0, keepdims=True)
    zn = (z - mean) * lax.rsqrt(var + 1e-5) * g_ref[...] + beta_ref[...]
    h3 = _SELU_SCALE * jnp.where(
        zn > 0.0, zn, _SELU_ALPHA * (jnp.exp(jnp.minimum(zn, 0.0)) - 1.0))
    h3_ref[...] = h3
    xt_ref[...] = dinv * h3


def _out_body(h3_ref, p_ref, dinv_ref, w2_ref, b2_ref, o_ref):
    h4 = h3_ref[...] + dinv_ref[...] * (p_ref[0] + p_ref[1])
    z = jnp.dot(h4, w2_ref[...], preferred_element_type=jnp.float32,
                precision=lax.Precision.HIGHEST) + b2_ref[...]
    z = z - jnp.max(z, axis=1, keepdims=True)
    ez = jnp.exp(z)
    o_ref[...] = ez / jnp.sum(ez, axis=1, keepdims=True)


def kernel(x, edge_index, W1, b1, gamma, beta, W2, b2):
    edges4 = edge_index.astype(jnp.int32).reshape(2, NW, NCHUNK, CH)

    f32 = jnp.float32
    hist = _hist_kernel(edges4)

    dinv, xt1 = pl.pallas_call(
        _prep_body,
        out_shape=(jax.ShapeDtypeStruct((N, 1), f32),
                   jax.ShapeDtypeStruct((N, F), f32)),
    )(hist, x)

    p1 = _prop_kernel(xt1, edges4)
    h1, xt2 = pl.pallas_call(
        _merge_body,
        out_shape=(jax.ShapeDtypeStruct((N, F), f32),
                   jax.ShapeDtypeStruct((N, F), f32)),
    )(x, p1, dinv)

    p2 = _prop_kernel(xt2, edges4)
    h3, xt3 = pl.pallas_call(
        _mid_body,
        out_shape=(jax.ShapeDtypeStruct((N, F), f32),
                   jax.ShapeDtypeStruct((N, F), f32)),
    )(h1, p2, dinv, W1, b1.reshape(1, F), gamma.reshape(1, F),
      beta.reshape(1, F))

    p3 = _prop_kernel(xt3, edges4)
    out = pl.pallas_call(
        _out_body,
        out_shape=jax.ShapeDtypeStruct((N, F), f32),
    )(h3, p3, dinv, W2, b2.reshape(1, F))
    return out


# speedup vs baseline: 29.5410x; 1.0148x over previous
import functools

import jax
import jax.numpy as jnp
from jax import lax
from jax.experimental import pallas as pl
from jax.experimental.pallas import tpu as pltpu
from jax.experimental.pallas import tpu_sc as plsc

N = 10000
E = 320000
F = 128

NC = 2
NS = 16
NW = NC * NS
EPW = E // NW
CH = 125
NCHUNK = EPW // CH
HALF = NCHUNK // 2
RPT = 624

_SELU_ALPHA = 1.6732632423543772
_SELU_SCALE = 1.0507009873554805

_mesh = plsc.VectorSubcoreMesh(core_axis_name="c", subcore_axis_name="s")
_sc_params = pltpu.CompilerParams(needs_layout_passes=False)


def _zero_rows(buf, rows):
    zeros = jnp.zeros((16,), jnp.float32)

    def body(i, _):
        r = i // (F // 16)
        l = i % (F // 16)
        buf[r, pl.ds(l * 16, 16)] = zeros
        return 0

    lax.fori_loop(0, rows * (F // 16), body, 0)


@functools.partial(
    pl.kernel,
    out_type=jax.ShapeDtypeStruct((NW, 1, N), jnp.float32),
    mesh=_mesh,
    compiler_params=_sc_params,
    scratch_types=[
        pltpu.VMEM((NCHUNK, CH), jnp.int32),
        pltpu.VMEM((N,), jnp.float32),
    ],
)
def _hist_kernel(edges_hbm, out_hbm, dst_v, hist_v):
    c = lax.axis_index("c")
    s = lax.axis_index("s")
    wid = s * NC + c
    pltpu.sync_copy(edges_hbm.at[1, wid], dst_v)

    zeros = jnp.zeros((16,), jnp.float32)

    def zbody(i, _):
        hist_v[pl.ds(i * 16, 16)] = zeros
        return 0

    lax.fori_loop(0, N // 16, zbody, 0)

    ones = jnp.ones((16,), jnp.float32)
    tail_mask = lax.iota(jnp.int32, 16) >= ((CH // 16) + 1) * 16 - CH

    def body(i, _):
        for l in range(CH // 16):
            idx = dst_v[i, pl.ds(l * 16, 16)]
            plsc.addupdate_scatter(hist_v, [idx], ones)
        idx = dst_v[i, pl.ds(CH - 16, 16)]
        plsc.addupdate_scatter(hist_v, [idx], ones, mask=tail_mask)
        return 0

    lax.fori_loop(0, NCHUNK, body, 0)
    pltpu.sync_copy(hist_v, out_hbm.at[wid, 0])


@functools.partial(
    pl.kernel,
    out_type=jax.ShapeDtypeStruct((NC, N, F), jnp.float32),
    mesh=_mesh,
    compiler_params=_sc_params,
    scratch_types=[
        pltpu.VMEM((HALF, CH), jnp.int32),
        pltpu.VMEM((HALF, CH), jnp.int32),
        pltpu.VMEM((CH, F), jnp.float32),
        pltpu.VMEM((CH, F), jnp.float32),
        pltpu.SemaphoreType.DMA,
        pltpu.SemaphoreType.DMA,
        pltpu.VMEM_SHARED((N, F), jnp.float32),
    ],
)
def _prop_kernel(xt_hbm, edges_hbm, out_hbm,
                 src_v, dst_v, rows_a, rows_b, sem_a, sem_b, acc):
    c = lax.axis_index("c")
    s = lax.axis_index("s")
    wid = s * NC + c

    pltpu.async_copy(edges_hbm.at[0, wid, pl.ds(0, HALF)], src_v, sem_a)
    pltpu.async_copy(edges_hbm.at[1, wid, pl.ds(0, HALF)], dst_v, sem_b)

    _zero_rows(rows_a, CH)
    base = s * RPT
    for t in range(5):
        pltpu.sync_copy(rows_a.at[pl.ds(0, 120)],
                        acc.at[pl.ds(base + t * 120, 120)])
    pltpu.sync_copy(rows_a.at[pl.ds(0, 24)], acc.at[pl.ds(base + 600, 24)])

    @pl.when(s == NS - 1)
    def _():
        pltpu.sync_copy(rows_a.at[pl.ds(0, N - NS * RPT)],
                        acc.at[pl.ds(NS * RPT, N - NS * RPT)])

    pltpu.make_async_copy(edges_hbm.at[0, wid, pl.ds(0, HALF)],
                          src_v, sem_a).wait()
    pltpu.make_async_copy(edges_hbm.at[1, wid, pl.ds(0, HALF)],
                          dst_v, sem_b).wait()
    plsc.subcore_barrier()

    bufs = (rows_a, rows_b)
    sems = (sem_a, sem_b)

    for r in range(2):
        if r == 1:
            pltpu.sync_copy(edges_hbm.at[0, wid, pl.ds(HALF, HALF)], src_v)
            pltpu.sync_copy(edges_hbm.at[1, wid, pl.ds(HALF, HALF)], dst_v)
        pltpu.async_copy(xt_hbm.at[src_v.at[0]], rows_a, sem_a)
        pltpu.async_copy(xt_hbm.at[src_v.at[1]], rows_b, sem_b)

        def pair(k, _):
            for b in range(2):
                j = 2 * k + b
                pltpu.make_async_copy(xt_hbm.at[src_v.at[j]],
                                      bufs[b], sems[b]).wait()
                pltpu.sync_copy(bufs[b], acc.at[dst_v.at[j]], add=True)
                pltpu.async_copy(xt_hbm.at[src_v.at[j + 2]], bufs[b], sems[b])
            return 0

        lax.fori_loop(0, HALF // 2 - 1, pair, 0)
        for b in range(2):
            j = HALF - 2 + b
            pltpu.make_async_copy(xt_hbm.at[src_v.at[j]],
                                  bufs[b], sems[b]).wait()
            pltpu.sync_copy(bufs[b], acc.at[dst_v.at[j]], add=True)

    plsc.subcore_barrier()

    pltpu.sync_copy(acc.at[pl.ds(base, RPT)], out_hbm.at[c, pl.ds(base, RPT)])

    @pl.when(s == NS - 1)
    def _():
        pltpu.sync_copy(acc.at[pl.ds(NS * RPT, N - NS * RPT)],
                        out_hbm.at[c, pl.ds(NS * RPT, N - NS * RPT)])


def _prep_body(hist_ref, x_ref, dinv_ref, xt_ref):
    deg = jnp.sum(hist_ref[:, 0, :], axis=0, keepdims=True)
    dinv_row = jnp.where(deg > 0.0, lax.rsqrt(jnp.maximum(deg, 1e-12)), 0.0)
    dinv = jnp.transpose(dinv_row)
    dinv_ref[...] = dinv
    xt_ref[...] = x_ref[...] * dinv


def _merge_body(x_ref, p_ref, dinv_ref, h_ref, xt_ref):
    dinv = dinv_ref[...]
    h = x_ref[...] + dinv * (p_ref[0] + p_ref[1])
    h_ref[...] = h
    xt_ref[...] = dinv * h


def _mid_body(h1_ref, p_ref, dinv_ref, w1_ref, b1_ref, g_ref, beta_ref,
              h3_ref, xt_ref):
    dinv = dinv_ref[...]
    h2 = h1_ref[...] + dinv * (p_ref[0] + p_ref[1])
    z = jnp.dot(h2, w1_ref[...], preferred_element_type=jnp.float32,
                precision=lax.Precision.HIGHEST) + b1_ref[...]
    mean = jnp.mean(z, axis=0, keepdims=True)
    var = jnp.mean(jnp.square(z - mean), axis=0, keepdims=True)
    zn = (z - mean) * lax.rsqrt(var + 1e-5) * g_ref[...] + beta_ref[...]
    h3 = _SELU_SCALE * jnp.where(
        zn > 0.0, zn, _SELU_ALPHA * (jnp.exp(jnp.minimum(zn, 0.0)) - 1.0))
    h3_ref[...] = h3
    xt_ref[...] = dinv * h3


def _out_body(h3_ref, p_ref, dinv_ref, w2_ref, b2_ref, o_ref):
    h4 = h3_ref[...] + dinv_ref[...] * (p_ref[0] + p_ref[1])
    z = jnp.dot(h4, w2_ref[...], preferred_element_type=jnp.float32,
                precision=lax.Precision.HIGHEST) + b2_ref[...]
    z = z - jnp.max(z, axis=1, keepdims=True)
    ez = jnp.exp(z)
    o_ref[...] = ez / jnp.sum(ez, axis=1, keepdims=True)


def kernel(x, edge_index, W1, b1, gamma, beta, W2, b2):
    edges4 = edge_index.astype(jnp.int32).reshape(2, NW, NCHUNK, CH)

    f32 = jnp.float32
    hist = _hist_kernel(edges4)

    dinv, xt1 = pl.pallas_call(
        _prep_body,
        out_shape=(jax.ShapeDtypeStruct((N, 1), f32),
                   jax.ShapeDtypeStruct((N, F), f32)),
    )(hist, x)

    p1 = _prop_kernel(xt1, edges4)
    h1, xt2 = pl.pallas_call(
        _merge_body,
        out_shape=(jax.ShapeDtypeStruct((N, F), f32),
                   jax.ShapeDtypeStruct((N, F), f32)),
    )(x, p1, dinv)

    p2 = _prop_kernel(xt2, edges4)
    h3, xt3 = pl.pallas_call(
        _mid_body,
        out_shape=(jax.ShapeDtypeStruct((N, F), f32),
                   jax.ShapeDtypeStruct((N, F), f32)),
    )(h1, p2, dinv, W1, b1.reshape(1, F), gamma.reshape(1, F),
      beta.reshape(1, F))

    p3 = _prop_kernel(xt3, edges4)
    out = pl.pallas_call(
        _out_body,
        out_shape=jax.ShapeDtypeStruct((N, F), f32),
    )(h3, p3, dinv, W2, b2.reshape(1, F))
    return out
